# baseline (speedup 1.0000x reference)
import jax
import jax.numpy as jnp
from jax.experimental import pallas as pl
from jax.experimental.pallas import tpu as pltpu

CL = 4096
CF = 304
KG = 112
HID = 256
TAU = 0.8
BM = 512
NB = CL // BM
KC = 256


def _mlp_norm(z, w1, b1, w2, b2):
    h = jnp.dot(z.astype(jnp.bfloat16), w1.astype(jnp.bfloat16),
                preferred_element_type=jnp.float32) + b1
    h = jnp.where(h > 0, h, jnp.exp(h) - 1.0)
    p = jnp.dot(h.astype(jnp.bfloat16), w2.astype(jnp.bfloat16),
                preferred_element_type=jnp.float32) + b2
    inv_norm = jax.lax.rsqrt(jnp.sum(p * p, axis=1, keepdims=True))
    return p * inv_norm


def _fused_kernel(z1_ref, z2_ref, w1cf_ref, b1cf_ref, w2cf_ref, b2cf_ref,
                  w1kg_ref, b1kg_ref, w2kg_ref, b2kg_ref, out_ref, p2_scr):
    i = pl.program_id(0)

    @pl.when(i == 0)
    def _():
        p2 = _mlp_norm(z2_ref[...], w1kg_ref[...], b1kg_ref[...],
                       w2kg_ref[...], b2kg_ref[...])
        p2_scr[...] = p2.astype(jnp.bfloat16)

    p1 = _mlp_norm(z1_ref[...], w1cf_ref[...], b1cf_ref[...],
                   w2cf_ref[...], b2cf_ref[...]) * (1.0 / TAU)

    p1b = p1.astype(jnp.bfloat16)

    def _tile(k):
        sk = jax.lax.dot_general(
            p1b, p2_scr[k * KC:(k + 1) * KC, :], (((1,), (1,)), ((), ())),
            preferred_element_type=jnp.float32,
        )
        return jnp.exp(sk)

    acc = _tile(0)
    for k in range(1, CL // KC):
        acc = acc + _tile(k)
    rowsum = jnp.sum(acc, axis=1)

    diag_blk = p2_scr[pl.ds(i * BM, BM), :].astype(jnp.float32)
    diag = jnp.sum(p1 * diag_blk, axis=1)
    partial = jnp.sum(jnp.log(rowsum + 1e-8) - diag) * (1.0 / CL)

    @pl.when(i == 0)
    def _():
        out_ref[0, 0] = 0.0

    out_ref[0, 0] += partial


def kernel(z1, z2, W1_cf, b1_cf, W2_cf, b2_cf, W1_kg, b1_kg, W2_kg, b2_kg):
    const = lambda i: (0, 0)
    out = pl.pallas_call(
        _fused_kernel,
        grid=(NB,),
        in_specs=[
            pl.BlockSpec((BM, CF), lambda i: (i, 0)),
            pl.BlockSpec((CL, KG), const),
            pl.BlockSpec((CF, HID), const),
            pl.BlockSpec((1, HID), const),
            pl.BlockSpec((HID, HID), const),
            pl.BlockSpec((1, HID), const),
            pl.BlockSpec((KG, HID), const),
            pl.BlockSpec((1, HID), const),
            pl.BlockSpec((HID, HID), const),
            pl.BlockSpec((1, HID), const),
        ],
        out_specs=pl.BlockSpec(memory_space=pltpu.SMEM),
        out_shape=jax.ShapeDtypeStruct((1, 1), jnp.float32),
        scratch_shapes=[pltpu.VMEM((CL, HID), jnp.bfloat16)],
    )(z1, z2, W1_cf, b1_cf.reshape(1, HID), W2_cf, b2_cf.reshape(1, HID),
      W1_kg, b1_kg.reshape(1, HID), W2_kg, b2_kg.reshape(1, HID))
    return out[0, 0]


# speedup vs baseline: 1.1244x; 1.1244x over previous
import jax
import jax.numpy as jnp
from jax.experimental import pallas as pl
from jax.experimental.pallas import tpu as pltpu

CL = 4096
CF = 304
KG = 112
HID = 256
TAU = 0.8
BM = 2048
NB = CL // BM
CHUNK = 128
LOG2E = 1.4426950408889634
LN2 = 0.6931471805599453


def _mlp_norm(z, w1, b1, w2, b2):
    h = jnp.dot(z.astype(jnp.bfloat16), w1.astype(jnp.bfloat16),
                preferred_element_type=jnp.float32) + b1
    h = jnp.where(h > 0, h, jnp.exp(h) - 1.0)
    p = jnp.dot(h.astype(jnp.bfloat16), w2.astype(jnp.bfloat16),
                preferred_element_type=jnp.float32) + b2
    inv_norm = jax.lax.rsqrt(jnp.sum(p * p, axis=1, keepdims=True))
    return p * inv_norm


def _fused_kernel(z1_ref, z2_ref, w1cf_ref, b1cf_ref, w2cf_ref, b2cf_ref,
                  w1kg_ref, b1kg_ref, w2kg_ref, b2kg_ref, out_ref, p2_scr):
    i = pl.program_id(0)

    @pl.when(i == 0)
    def _():
        p2 = _mlp_norm(z2_ref[...], w1kg_ref[...], b1kg_ref[...],
                       w2kg_ref[...], b2kg_ref[...])
        p2_scr[...] = p2.astype(jnp.bfloat16)

    p1 = _mlp_norm(z1_ref[...], w1cf_ref[...], b1cf_ref[...],
                   w2cf_ref[...], b2cf_ref[...]) * (LOG2E / TAU)

    u = jax.lax.dot_general(
        p1.astype(jnp.bfloat16), p2_scr[...], (((1,), (1,)), ((), ())),
        preferred_element_type=jnp.float32,
    )
    acc = jax.lax.exp2(u[:, :CHUNK])
    for k in range(1, CL // CHUNK):
        acc = acc + jax.lax.exp2(u[:, k * CHUNK:(k + 1) * CHUNK])
    rowsum = jnp.sum(acc, axis=1)

    diag_blk = p2_scr[pl.ds(i * BM, BM), :].astype(jnp.float32)
    diag = jnp.sum(p1 * diag_blk, axis=1) * LN2
    partial = jnp.sum(jnp.log(rowsum + 1e-8) - diag) * (1.0 / CL)

    @pl.when(i == 0)
    def _():
        out_ref[0, 0] = 0.0

    out_ref[0, 0] += partial


def kernel(z1, z2, W1_cf, b1_cf, W2_cf, b2_cf, W1_kg, b1_kg, W2_kg, b2_kg):
    const = lambda i: (0, 0)
    out = pl.pallas_call(
        _fused_kernel,
        grid=(NB,),
        in_specs=[
            pl.BlockSpec((BM, CF), lambda i: (i, 0)),
            pl.BlockSpec((CL, KG), const),
            pl.BlockSpec((CF, HID), const),
            pl.BlockSpec((1, HID), const),
            pl.BlockSpec((HID, HID), const),
            pl.BlockSpec((1, HID), const),
            pl.BlockSpec((KG, HID), const),
            pl.BlockSpec((1, HID), const),
            pl.BlockSpec((HID, HID), const),
            pl.BlockSpec((1, HID), const),
        ],
        out_specs=pl.BlockSpec(memory_space=pltpu.SMEM),
        out_shape=jax.ShapeDtypeStruct((1, 1), jnp.float32),
        scratch_shapes=[pltpu.VMEM((CL, HID), jnp.bfloat16)],
    )(z1, z2, W1_cf, b1_cf.reshape(1, HID), W2_cf, b2_cf.reshape(1, HID),
      W1_kg, b1_kg.reshape(1, HID), W2_kg, b2_kg.reshape(1, HID))
    return out[0, 0]
